# baseline (speedup 1.0000x reference)
import functools
import math

import jax
import jax.numpy as jnp
from jax import lax
from jax.experimental import pallas as pl
from jax.experimental.pallas import tpu as pltpu
from jax.experimental.pallas import tpu_sc as plsc

VOCAB = 100000
D = 768
K = 128
SPARSITY = 32
EPS = 0.05
N_ITER = 50
B = 2
S = 2048

CH = 512
NCH = S // CH


def _make_sc_gather():
    info = plsc.get_sparse_core_info()
    nw = info.num_cores * info.num_subcores
    n_tok = B * S
    assert n_tok % (8 * nw) == 0
    b_per_w = n_tok // nw
    mesh = plsc.VectorSubcoreMesh(core_axis_name="c", subcore_axis_name="s")

    @functools.partial(
        pl.kernel,
        mesh=mesh,
        out_type=jax.ShapeDtypeStruct((n_tok, D), jnp.float32),
        scratch_types=[
            pltpu.VMEM((b_per_w,), jnp.int32),
            pltpu.VMEM((b_per_w, D), jnp.float32),
            pltpu.SemaphoreType.DMA,
        ],
    )
    def gather_kernel(table_hbm, idx_hbm, out_hbm, idx_v, rows_v, sem):
        wid = lax.axis_index("s") * info.num_cores + lax.axis_index("c")
        base = wid * b_per_w
        pltpu.sync_copy(idx_hbm.at[pl.ds(base, b_per_w)], idx_v)
        pltpu.async_copy(table_hbm.at[idx_v], rows_v, sem).wait()
        pltpu.sync_copy(rows_v, out_hbm.at[pl.ds(base, b_per_w)])

    return gather_kernel


def _tc_body(x_ref, proto_ref, w_ref, b_ref, zr_ref, zi_ref, c_ref, f_ref):
    inv_eps = 1.0 / EPS
    log_mu = -math.log(S)
    log_nu = -math.log(K)

    proto = proto_ref[:]
    ones_row = jnp.ones((1, D), jnp.float32)
    p2 = lax.dot_general(ones_row, proto * proto,
                         (((1,), (1,)), ((), ())),
                         preferred_element_type=jnp.float32)

    for j in range(NCH):
        xc = x_ref[0, pl.ds(j * CH, CH), :]
        xp = lax.dot_general(xc, proto, (((1,), (1,)), ((), ())),
                             preferred_element_type=jnp.float32)
        x2 = jnp.sum(xc * xc, axis=1, keepdims=True)
        c_ref[pl.ds(j * CH, CH), :] = (x2 + p2 - 2.0 * xp) / D

    def sink_body(_, g):
        m_run = jnp.full((1, K), -1e30, jnp.float32)
        s_run = jnp.zeros((1, K), jnp.float32)
        for j in range(NCH):
            cc = c_ref[pl.ds(j * CH, CH), :]
            a = (g - cc) * inv_eps
            am = jnp.max(a, axis=1, keepdims=True)
            lse = jnp.log(jnp.sum(jnp.exp(a - am), axis=1, keepdims=True)) + am
            fc = EPS * (log_mu - lse)
            f_ref[pl.ds(j * CH, CH), :] = fc
            bb = (fc - cc) * inv_eps
            bm = jnp.max(bb, axis=0, keepdims=True)
            m_new = jnp.maximum(m_run, bm)
            s_run = s_run * jnp.exp(m_run - m_new) + jnp.sum(
                jnp.exp(bb - m_new), axis=0, keepdims=True)
            m_run = m_new
        return EPS * (log_nu - (jnp.log(s_run) + m_run))

    g = lax.fori_loop(0, N_ITER, sink_body, jnp.zeros((1, K), jnp.float32))

    lane = lax.broadcasted_iota(jnp.int32, (CH, K), 1)
    dcol = lax.broadcasted_iota(jnp.int32, (CH, D), 1).astype(jnp.float32)
    srow = lax.broadcasted_iota(jnp.int32, (CH, D), 0).astype(jnp.float32)
    div_term = jnp.exp(dcol * (-math.log(10000.0) / D))
    bias = b_ref[:]

    for j in range(NCH):
        cc = c_ref[pl.ds(j * CH, CH), :]
        fc = f_ref[pl.ds(j * CH, CH), :]
        t = jnp.exp((fc + g - cc) * inv_eps)

        def ext_body(_, carry):
            vals, mask, tau = carry
            m = jnp.max(vals, axis=1, keepdims=True)
            eq = vals == m
            idx = jnp.min(jnp.where(eq, lane, K), axis=1, keepdims=True)
            sel = lane == idx
            mask = jnp.where(sel, 1.0, mask)
            vals = jnp.where(sel, -jnp.inf, vals)
            return vals, mask, m

        _, mask, tau = lax.fori_loop(
            0, SPARSITY, ext_body,
            (t, jnp.zeros((CH, K), jnp.float32), jnp.zeros((CH, 1), jnp.float32)))

        t_sparse = jnp.maximum(t - tau, 0.0)
        fb = jnp.sum(t_sparse, axis=1, keepdims=True) <= 1e-12
        t_fin = jnp.where(fb, t * mask, t_sparse)

        sdr = lax.dot_general(t_fin, w_ref[:], (((1,), (1,)), ((), ())),
                              preferred_element_type=jnp.float32) + bias
        phase = (srow + jnp.float32(j * CH)) * div_term
        zr_ref[0, pl.ds(j * CH, CH), :] = sdr * jnp.cos(phase)
        zi_ref[0, pl.ds(j * CH, CH), :] = sdr * jnp.sin(phase)


def _tc_call(x, proto, w_out, b_out):
    return pl.pallas_call(
        _tc_body,
        grid=(B,),
        in_specs=[
            pl.BlockSpec((1, S, D), lambda b: (b, 0, 0)),
            pl.BlockSpec((K, D), lambda b: (0, 0)),
            pl.BlockSpec((D, K), lambda b: (0, 0)),
            pl.BlockSpec((1, D), lambda b: (0, 0)),
        ],
        out_specs=[
            pl.BlockSpec((1, S, D), lambda b: (b, 0, 0)),
            pl.BlockSpec((1, S, D), lambda b: (b, 0, 0)),
        ],
        out_shape=[
            jax.ShapeDtypeStruct((B, S, D), jnp.float32),
            jax.ShapeDtypeStruct((B, S, D), jnp.float32),
        ],
        scratch_shapes=[
            pltpu.VMEM((S, K), jnp.float32),
            pltpu.VMEM((S, 1), jnp.float32),
        ],
    )(x, proto, w_out, b_out)


def kernel(token_ids, emb_table, proto, W_out, b_out):
    idx = token_ids.astype(jnp.int32).reshape(B * S)
    x = _make_sc_gather()(emb_table, idx)
    x = x.reshape(B, S, D)
    zr, zi = _tc_call(x, proto, W_out, b_out.reshape(1, D))
    return jax.lax.complex(zr, zi)


# speedup vs baseline: 1.4564x; 1.4564x over previous
import functools
import math

import jax
import jax.numpy as jnp
from jax import lax
from jax.experimental import pallas as pl
from jax.experimental.pallas import tpu as pltpu
from jax.experimental.pallas import tpu_sc as plsc

VOCAB = 100000
D = 768
K = 128
SPARSITY = 32
EPS = 0.05
N_ITER = 50
B = 2
S = 2048

CH = 512
NCH = S // CH
SINK_TOL = 1e-5


def _make_sc_gather():
    info = plsc.get_sparse_core_info()
    nw = info.num_cores * info.num_subcores
    n_tok = B * S
    assert n_tok % (8 * nw) == 0
    b_per_w = n_tok // nw
    mesh = plsc.VectorSubcoreMesh(core_axis_name="c", subcore_axis_name="s")

    @functools.partial(
        pl.kernel,
        mesh=mesh,
        out_type=jax.ShapeDtypeStruct((n_tok, D), jnp.float32),
        scratch_types=[
            pltpu.VMEM((b_per_w,), jnp.int32),
            pltpu.VMEM((b_per_w, D), jnp.float32),
            pltpu.SemaphoreType.DMA,
        ],
    )
    def gather_kernel(table_hbm, idx_hbm, out_hbm, idx_v, rows_v, sem):
        wid = lax.axis_index("s") * info.num_cores + lax.axis_index("c")
        base = wid * b_per_w
        pltpu.sync_copy(idx_hbm.at[pl.ds(base, b_per_w)], idx_v)
        pltpu.async_copy(table_hbm.at[idx_v], rows_v, sem).wait()
        pltpu.sync_copy(rows_v, out_hbm.at[pl.ds(base, b_per_w)])

    return gather_kernel


def _tc_body(x_ref, proto_ref, w_ref, b_ref, zr_ref, zi_ref, ke_ref, u_ref):
    inv_eps = 1.0 / EPS
    mu = 1.0 / S
    nu = 1.0 / K

    proto = proto_ref[:]
    ones_row = jnp.ones((1, D), jnp.float32)
    p2 = lax.dot_general(ones_row, proto * proto,
                         (((1,), (1,)), ((), ())),
                         preferred_element_type=jnp.float32)

    for j in range(NCH):
        xc = x_ref[0, pl.ds(j * CH, CH), :]
        xp = lax.dot_general(xc, proto, (((1,), (1,)), ((), ())),
                             preferred_element_type=jnp.float32)
        x2 = jnp.sum(xc * xc, axis=1, keepdims=True)
        c = (x2 + p2 - 2.0 * xp) / D
        ke_ref[pl.ds(j * CH, CH), :] = jnp.exp(c * (-inv_eps))

    def sink_cond(carry):
        i, _, delta = carry
        return jnp.logical_and(i < N_ITER, delta > SINK_TOL)

    def sink_body(carry):
        i, w, _ = carry
        s_run = jnp.zeros((1, K), jnp.float32)
        for j in range(NCH):
            ke = ke_ref[pl.ds(j * CH, CH), :]
            r = jnp.sum(ke * w, axis=1, keepdims=True)
            uc = mu / r
            u_ref[pl.ds(j * CH, CH), :] = uc
            s_run = s_run + jnp.sum(ke * uc, axis=0, keepdims=True)
        w_new = nu / s_run
        delta = jnp.max(jnp.abs(w_new - w) / w_new)
        return i + 1, w_new, delta

    _, w, _ = lax.while_loop(
        sink_cond, sink_body,
        (jnp.int32(0), jnp.ones((1, K), jnp.float32), jnp.float32(jnp.inf)))

    lane = lax.broadcasted_iota(jnp.int32, (CH, K), 1)
    ri = lax.broadcasted_iota(jnp.int32, (K, K), 0)
    ci = lax.broadcasted_iota(jnp.int32, (K, K), 1)
    tri_strict = (ri < ci).astype(jnp.float32)

    wrow = jnp.exp(
        lax.broadcasted_iota(jnp.int32, (1, D), 1).astype(jnp.float32)
        * (-math.log(10000.0) / D))
    a8 = (lax.broadcasted_iota(jnp.int32, (64, D), 0) * 8).astype(jnp.float32)
    t8c = jnp.cos(a8 * wrow)
    t8s = jnp.sin(a8 * wrow)
    b8 = lax.broadcasted_iota(jnp.int32, (8, D), 0).astype(jnp.float32)
    b8c = jnp.cos(b8 * wrow)
    b8s = jnp.sin(b8 * wrow)
    b8c_e = jnp.broadcast_to(b8c[None], (64, 8, D)).reshape(CH, D)
    b8s_e = jnp.broadcast_to(b8s[None], (64, 8, D)).reshape(CH, D)

    bias = b_ref[:]

    for j in range(NCH):
        ke = ke_ref[pl.ds(j * CH, CH), :]
        uc = u_ref[pl.ds(j * CH, CH), :]
        t = uc * ke * w

        def ext_body(_, carry):
            vals, cnt, tau = carry
            m = jnp.max(vals, axis=1, keepdims=True)
            eqm = vals == m
            nc = cnt + jnp.sum(eqm.astype(jnp.float32), axis=1, keepdims=True)
            tau = jnp.where(
                jnp.logical_and(cnt < SPARSITY, nc >= SPARSITY), m, tau)
            vals = jnp.where(eqm, -jnp.inf, vals)
            return vals, nc, tau

        _, _, tau = lax.fori_loop(
            0, SPARSITY, ext_body,
            (t, jnp.zeros((CH, 1), jnp.float32), jnp.zeros((CH, 1), jnp.float32)))

        gt = t > tau
        eq = t == tau
        n_gt = jnp.sum(gt.astype(jnp.float32), axis=1, keepdims=True)
        excl = lax.dot_general(eq.astype(jnp.float32), tri_strict,
                               (((1,), (0,)), ((), ())),
                               preferred_element_type=jnp.float32)
        need = SPARSITY - n_gt
        sel = jnp.logical_or(gt, jnp.logical_and(eq, excl < need))
        mask = jnp.where(sel, 1.0, 0.0)

        t_sparse = jnp.maximum(t - tau, 0.0)
        fb = jnp.sum(t_sparse, axis=1, keepdims=True) <= 1e-12
        t_fin = jnp.where(fb, t * mask, t_sparse)

        sdr = lax.dot_general(t_fin, w_ref[:], (((1,), (1,)), ((), ())),
                              preferred_element_type=jnp.float32) + bias

        base = jnp.float32(j * CH)
        c0 = jnp.cos(base * wrow)
        s0 = jnp.sin(base * wrow)
        ca = c0 * t8c - s0 * t8s
        sa = s0 * t8c + c0 * t8s
        ca_e = jnp.broadcast_to(ca[:, None, :], (64, 8, D)).reshape(CH, D)
        sa_e = jnp.broadcast_to(sa[:, None, :], (64, 8, D)).reshape(CH, D)
        cosp = ca_e * b8c_e - sa_e * b8s_e
        sinp = sa_e * b8c_e + ca_e * b8s_e
        zr_ref[0, pl.ds(j * CH, CH), :] = sdr * cosp
        zi_ref[0, pl.ds(j * CH, CH), :] = sdr * sinp


def _tc_call(x, proto, w_out, b_out):
    return pl.pallas_call(
        _tc_body,
        grid=(B,),
        in_specs=[
            pl.BlockSpec((1, S, D), lambda b: (b, 0, 0)),
            pl.BlockSpec((K, D), lambda b: (0, 0)),
            pl.BlockSpec((D, K), lambda b: (0, 0)),
            pl.BlockSpec((1, D), lambda b: (0, 0)),
        ],
        out_specs=[
            pl.BlockSpec((1, S, D), lambda b: (b, 0, 0)),
            pl.BlockSpec((1, S, D), lambda b: (b, 0, 0)),
        ],
        out_shape=[
            jax.ShapeDtypeStruct((B, S, D), jnp.float32),
            jax.ShapeDtypeStruct((B, S, D), jnp.float32),
        ],
        scratch_shapes=[
            pltpu.VMEM((S, K), jnp.float32),
            pltpu.VMEM((S, 1), jnp.float32),
        ],
    )(x, proto, w_out, b_out)


def kernel(token_ids, emb_table, proto, W_out, b_out):
    idx = token_ids.astype(jnp.int32).reshape(B * S)
    x = _make_sc_gather()(emb_table, idx)
    x = x.reshape(B, S, D)
    zr, zi = _tc_call(x, proto, W_out, b_out.reshape(1, D))
    return jax.lax.complex(zr, zi)


# speedup vs baseline: 1.4582x; 1.0012x over previous
import functools
import math

import jax
import jax.numpy as jnp
from jax import lax
from jax.experimental import pallas as pl
from jax.experimental.pallas import tpu as pltpu
from jax.experimental.pallas import tpu_sc as plsc

VOCAB = 100000
D = 768
K = 128
SPARSITY = 32
EPS = 0.05
N_ITER = 50
B = 2
S = 2048

CH = 512
NCH = S // CH
SINK_TOL = 1e-5


def _make_sc_gather():
    info = plsc.get_sparse_core_info()
    nw = info.num_cores * info.num_subcores
    n_tok = B * S
    b_per_w = n_tok // nw
    mesh = plsc.VectorSubcoreMesh(core_axis_name="c", subcore_axis_name="s")

    @functools.partial(
        pl.kernel,
        mesh=mesh,
        out_type=jax.ShapeDtypeStruct((n_tok, D), jnp.float32),
        scratch_types=[
            pltpu.VMEM((b_per_w,), jnp.int32),
            pltpu.VMEM((b_per_w, D), jnp.float32),
            pltpu.SemaphoreType.DMA,
        ],
    )
    def gather_kernel(table_hbm, idx_hbm, out_hbm, idx_v, rows_v, sem):
        wid = lax.axis_index("s") * info.num_cores + lax.axis_index("c")
        base = wid * b_per_w
        pltpu.sync_copy(idx_hbm.at[pl.ds(base, b_per_w)], idx_v)
        pltpu.async_copy(table_hbm.at[idx_v], rows_v, sem).wait()
        pltpu.sync_copy(rows_v, out_hbm.at[pl.ds(base, b_per_w)])

    return gather_kernel


def _tc_body(x_ref, proto_ref, w_ref, b_ref, zr_ref, zi_ref, ke_ref, u_ref):
    inv_eps = 1.0 / EPS
    mu = 1.0 / S
    nu = 1.0 / K

    proto = proto_ref[:]
    ones_row = jnp.ones((1, D), jnp.float32)
    p2 = lax.dot_general(ones_row, proto * proto,
                         (((1,), (1,)), ((), ())),
                         preferred_element_type=jnp.float32)

    for j in range(NCH):
        xc = x_ref[0, pl.ds(j * CH, CH), :]
        xp = lax.dot_general(xc, proto, (((1,), (1,)), ((), ())),
                             preferred_element_type=jnp.float32)
        x2 = jnp.sum(xc * xc, axis=1, keepdims=True)
        c = (x2 + p2 - 2.0 * xp) / D
        ke_ref[pl.ds(j * CH, CH), :] = jnp.exp(c * (-inv_eps))

    def sink_cond(carry):
        i, _, delta = carry
        return jnp.logical_and(i < N_ITER, delta > SINK_TOL)

    def sink_body(carry):
        i, w, _ = carry
        s_run = jnp.zeros((1, K), jnp.float32)
        for j in range(NCH):
            ke = ke_ref[pl.ds(j * CH, CH), :]
            r = jnp.sum(ke * w, axis=1, keepdims=True)
            uc = mu / r
            u_ref[pl.ds(j * CH, CH), :] = uc
            s_run = s_run + jnp.sum(ke * uc, axis=0, keepdims=True)
        w_new = nu / s_run
        delta = jnp.max(jnp.abs(w_new - w) / w_new)
        return i + 1, w_new, delta

    _, w, _ = lax.while_loop(
        sink_cond, sink_body,
        (jnp.int32(0), jnp.ones((1, K), jnp.float32), jnp.float32(jnp.inf)))

    lane = lax.broadcasted_iota(jnp.int32, (CH, K), 1)
    ri = lax.broadcasted_iota(jnp.int32, (K, K), 0)
    ci = lax.broadcasted_iota(jnp.int32, (K, K), 1)
    tri_strict = (ri < ci).astype(jnp.float32)

    wrow = jnp.exp(
        lax.broadcasted_iota(jnp.int32, (1, D), 1).astype(jnp.float32)
        * (-math.log(10000.0) / D))
    a8 = (lax.broadcasted_iota(jnp.int32, (64, D), 0) * 8).astype(jnp.float32)
    t8c = jnp.cos(a8 * wrow)
    t8s = jnp.sin(a8 * wrow)
    b8 = lax.broadcasted_iota(jnp.int32, (8, D), 0).astype(jnp.float32)
    b8c = jnp.cos(b8 * wrow)
    b8s = jnp.sin(b8 * wrow)
    b8c_e = jnp.broadcast_to(b8c[None], (64, 8, D)).reshape(CH, D)
    b8s_e = jnp.broadcast_to(b8s[None], (64, 8, D)).reshape(CH, D)

    bias = b_ref[:]

    for j in range(NCH):
        ke = ke_ref[pl.ds(j * CH, CH), :]
        uc = u_ref[pl.ds(j * CH, CH), :]
        t = uc * ke * w

        def ext_body(_, carry):
            vals, cnt, tau = carry
            m = jnp.max(vals, axis=1, keepdims=True)
            eqm = vals == m
            nc = cnt + jnp.sum(eqm.astype(jnp.float32), axis=1, keepdims=True)
            tau = jnp.where(
                jnp.logical_and(cnt < SPARSITY, nc >= SPARSITY), m, tau)
            vals = jnp.where(eqm, -jnp.inf, vals)
            return vals, nc, tau

        _, _, tau = lax.fori_loop(
            0, SPARSITY, ext_body,
            (t, jnp.zeros((CH, 1), jnp.float32), jnp.zeros((CH, 1), jnp.float32)))

        gt = t > tau
        eq = t == tau
        n_gt = jnp.sum(gt.astype(jnp.float32), axis=1, keepdims=True)
        excl = lax.dot_general(eq.astype(jnp.float32), tri_strict,
                               (((1,), (0,)), ((), ())),
                               preferred_element_type=jnp.float32)
        need = SPARSITY - n_gt
        sel = jnp.logical_or(gt, jnp.logical_and(eq, excl < need))
        mask = jnp.where(sel, 1.0, 0.0)

        t_sparse = jnp.maximum(t - tau, 0.0)
        fb = jnp.sum(t_sparse, axis=1, keepdims=True) <= 1e-12
        t_fin = jnp.where(fb, t * mask, t_sparse)

        sdr = lax.dot_general(t_fin, w_ref[:], (((1,), (1,)), ((), ())),
                              preferred_element_type=jnp.float32) + bias

        base = jnp.float32(j * CH)
        c0 = jnp.cos(base * wrow)
        s0 = jnp.sin(base * wrow)
        ca = c0 * t8c - s0 * t8s
        sa = s0 * t8c + c0 * t8s
        ca_e = jnp.broadcast_to(ca[:, None, :], (64, 8, D)).reshape(CH, D)
        sa_e = jnp.broadcast_to(sa[:, None, :], (64, 8, D)).reshape(CH, D)
        cosp = ca_e * b8c_e - sa_e * b8s_e
        sinp = sa_e * b8c_e + ca_e * b8s_e
        zr_ref[0, pl.ds(j * CH, CH), :] = sdr * cosp
        zi_ref[0, pl.ds(j * CH, CH), :] = sdr * sinp


def _tc_call(x, proto, w_out, b_out):
    return pl.pallas_call(
        _tc_body,
        grid=(B,),
        in_specs=[
            pl.BlockSpec((1, S, D), lambda b: (b, 0, 0)),
            pl.BlockSpec((K, D), lambda b: (0, 0)),
            pl.BlockSpec((D, K), lambda b: (0, 0)),
            pl.BlockSpec((1, D), lambda b: (0, 0)),
        ],
        out_specs=[
            pl.BlockSpec((1, S, D), lambda b: (b, 0, 0)),
            pl.BlockSpec((1, S, D), lambda b: (b, 0, 0)),
        ],
        out_shape=[
            jax.ShapeDtypeStruct((B, S, D), jnp.float32),
            jax.ShapeDtypeStruct((B, S, D), jnp.float32),
        ],
        scratch_shapes=[
            pltpu.VMEM((S, K), jnp.float32),
            pltpu.VMEM((S, 1), jnp.float32),
        ],
    )(x, proto, w_out, b_out)


def kernel(token_ids, emb_table, proto, W_out, b_out):
    idx = token_ids.astype(jnp.int32).reshape(B * S)
    x = _make_sc_gather()(emb_table, idx)
    x = x.reshape(B, S, D)
    zr, zi = _tc_call(x, proto, W_out, b_out.reshape(1, D))
    return jax.lax.complex(zr, zi)
